# baseline (speedup 1.0000x reference)
import functools

import jax
import jax.numpy as jnp
from jax import lax
from jax.experimental import pallas as pl
from jax.experimental.pallas import tpu as pltpu
from jax.experimental.pallas import tpu_sc as plsc

_D = 128
_T = 128
_NC = 2
_NS = 16
_NW = _NC * _NS
_CH = 128
_K = 2


@functools.partial(jax.jit, static_argnums=0)
def _gather(B, idx2d, tbl):
    n_ch = B // (_NW * _CH)

    def body(idx_hbm, table_hbm, out_hbm, idx_v, table_v, rows_v, zrow_v, gsem, osem):
        wid = lax.axis_index("s") * _NC + lax.axis_index("c")
        row0 = wid * n_ch

        @pl.when(lax.axis_index("s") == 0)
        def _():
            pltpu.sync_copy(table_hbm, table_v)
            for j in range(_D // 16):
                zrow_v[0, pl.ds(j * 16, 16)] = jnp.zeros((16,), jnp.float32)
            pltpu.sync_copy(zrow_v, table_v.at[pl.ds(0, 1), :])

        pltpu.sync_copy(idx_hbm.at[pl.ds(row0, n_ch), :], idx_v)
        plsc.subcore_barrier()

        n_it = n_ch // _K

        def launch(it, b):
            for j in range(_K):
                pltpu.async_copy(
                    table_v.at[idx_v.at[it * _K + j]],
                    rows_v.at[b, pl.ds(j * _CH, _CH)],
                    gsem,
                )

        def drain(it, b):
            for j in range(_K):
                pltpu.make_async_copy(
                    table_v.at[idx_v.at[it * _K + j]],
                    rows_v.at[b, pl.ds(j * _CH, _CH)],
                    gsem,
                ).wait()

        def ocopy(it, b):
            return pltpu.make_async_copy(
                rows_v.at[b],
                out_hbm.at[pl.ds((row0 + it * _K) * _CH, _K * _CH), :],
                osem,
            )

        launch(0, 0)

        def step(it, carry):
            b = lax.rem(it, 3)
            drain(it, b)
            @pl.when(it >= 2)
            def _():
                ocopy(it - 2, lax.rem(it - 2, 3)).wait()

            launch(jnp.minimum(it + 1, n_it - 1), lax.rem(it + 1, 3))
            ocopy(it, b).start()
            return carry

        lax.fori_loop(0, n_it, step, 0)
        drain(n_it - 1, lax.rem(n_it, 3))
        for k in (2, 1):
            it = n_it - k
            ocopy(it, it % 3).wait()

    mesh = plsc.VectorSubcoreMesh(core_axis_name="c", subcore_axis_name="s")
    f = pl.kernel(
        body,
        out_type=jax.ShapeDtypeStruct((B, _D), jnp.float32),
        mesh=mesh,
        scratch_types=[
            pltpu.VMEM((n_ch, _CH), jnp.int32),
            pltpu.VMEM_SHARED((_T, _D), jnp.float32),
            pltpu.VMEM((3, _K * _CH, _D), jnp.float32),
            pltpu.VMEM((1, _D), jnp.float32),
            pltpu.SemaphoreType.DMA,
            pltpu.SemaphoreType.DMA,
        ],
    )
    return f(idx2d, tbl)


def kernel(Z, table):
    n, m = Z.shape
    B = n * m
    idx2d = Z.reshape(B // _CH, _CH).astype(jnp.int32)
    out = _gather(B, idx2d, table)
    return out.reshape(n, m, _D)


# speedup vs baseline: 1.0358x; 1.0358x over previous
import functools

import jax
import jax.numpy as jnp
from jax import lax
from jax.experimental import pallas as pl
from jax.experimental.pallas import tpu as pltpu
from jax.experimental.pallas import tpu_sc as plsc

_D = 128
_T = 128
_NC = 2
_NS = 16
_NW = _NC * _NS
_CH = 128


@functools.partial(jax.jit, static_argnums=0)
def _gather(B, idx2d, tbl):
    n_ch = B // (_NW * _CH)

    def body(idx_hbm, table_hbm, out_hbm, idx_v, table_v, rows_v, zrow_v, gsem, osem):
        wid = lax.axis_index("s") * _NC + lax.axis_index("c")
        row0 = wid * n_ch

        @pl.when(lax.axis_index("s") == 0)
        def _():
            pltpu.sync_copy(table_hbm, table_v)
            for j in range(_D // 16):
                zrow_v[0, pl.ds(j * 16, 16)] = jnp.zeros((16,), jnp.float32)
            pltpu.sync_copy(zrow_v, table_v.at[pl.ds(0, 1), :])

        pltpu.sync_copy(idx_hbm.at[pl.ds(row0, n_ch), :], idx_v)
        plsc.subcore_barrier()

        def gcopy(g, b):
            return pltpu.make_async_copy(
                table_v.at[idx_v.at[g]], rows_v.at[b], gsem
            )

        def ocopy(g, b):
            return pltpu.make_async_copy(
                rows_v.at[b],
                out_hbm.at[pl.ds((row0 + g) * _CH, _CH), :],
                osem,
            )

        for g0 in range(3):
            gcopy(g0, g0).start()

        def chunk(g, carry):
            b = lax.rem(g, 4)
            gcopy(g, b).wait()
            ocopy(g, b).start()
            @pl.when(g >= 1)
            def _():
                ocopy(g - 1, lax.rem(g - 1, 4)).wait()

            gcopy(jnp.minimum(g + 3, n_ch - 1), lax.rem(g + 3, 4)).start()
            return carry

        lax.fori_loop(0, n_ch, chunk, 0)
        for _ in range(3):
            gcopy(n_ch - 1, 3).wait()
        ocopy(n_ch - 1, lax.rem(n_ch - 1, 4)).wait()

    mesh = plsc.VectorSubcoreMesh(core_axis_name="c", subcore_axis_name="s")
    f = pl.kernel(
        body,
        out_type=jax.ShapeDtypeStruct((B, _D), jnp.float32),
        mesh=mesh,
        scratch_types=[
            pltpu.VMEM((n_ch, _CH), jnp.int32),
            pltpu.VMEM_SHARED((_T, _D), jnp.float32),
            pltpu.VMEM((4, _CH, _D), jnp.float32),
            pltpu.VMEM((1, _D), jnp.float32),
            pltpu.SemaphoreType.DMA,
            pltpu.SemaphoreType.DMA,
        ],
    )
    return f(idx2d, tbl)


def kernel(Z, table):
    n, m = Z.shape
    B = n * m
    idx2d = Z.reshape(B // _CH, _CH).astype(jnp.int32)
    out = _gather(B, idx2d, table)
    return out.reshape(n, m, _D)


# speedup vs baseline: 1.0367x; 1.0008x over previous
import functools

import jax
import jax.numpy as jnp
from jax import lax
from jax.experimental import pallas as pl
from jax.experimental.pallas import tpu as pltpu
from jax.experimental.pallas import tpu_sc as plsc

_D = 128
_T = 128
_NC = 2
_NS = 16
_NW = _NC * _NS
_CH = 128


@functools.partial(jax.jit, static_argnums=0)
def _gather(B, idx2d, tbl):
    n_ch = B // (_NW * _CH)

    def body(idx_hbm, table_hbm, out_hbm, idx_v, table_v, rows_v, zrow_v, gsem, osem):
        wid = lax.axis_index("s") * _NC + lax.axis_index("c")
        row0 = wid * n_ch

        @pl.when(lax.axis_index("s") == 0)
        def _():
            pltpu.sync_copy(table_hbm, table_v)
            for j in range(_D // 16):
                zrow_v[0, pl.ds(j * 16, 16)] = jnp.zeros((16,), jnp.float32)
            pltpu.sync_copy(zrow_v, table_v.at[pl.ds(0, 1), :])

        pltpu.sync_copy(idx_hbm.at[pl.ds(row0, n_ch), :], idx_v)
        plsc.subcore_barrier()

        def gcopy(g, b):
            return pltpu.make_async_copy(
                table_v.at[idx_v.at[g]], rows_v.at[b], gsem
            )

        def ocopy(g, b):
            return pltpu.make_async_copy(
                rows_v.at[b],
                out_hbm.at[pl.ds((row0 + g) * _CH, _CH), :],
                osem,
            )

        for g0 in range(3):
            gcopy(g0, g0).start()

        def chunk(g, carry):
            b = lax.rem(g, 6)
            gcopy(g, b).wait()
            ocopy(g, b).start()
            @pl.when(g >= 3)
            def _():
                ocopy(g - 3, lax.rem(g - 3, 6)).wait()

            gcopy(jnp.minimum(g + 3, n_ch - 1), lax.rem(g + 3, 6)).start()
            return carry

        lax.fori_loop(0, n_ch, chunk, 0)
        for _ in range(3):
            gcopy(n_ch - 1, 0).wait()
        for k in (3, 2, 1):
            g = n_ch - k
            ocopy(g, g % 6).wait()

    mesh = plsc.VectorSubcoreMesh(core_axis_name="c", subcore_axis_name="s")
    f = pl.kernel(
        body,
        out_type=jax.ShapeDtypeStruct((B, _D), jnp.float32),
        mesh=mesh,
        scratch_types=[
            pltpu.VMEM((n_ch, _CH), jnp.int32),
            pltpu.VMEM_SHARED((_T, _D), jnp.float32),
            pltpu.VMEM((6, _CH, _D), jnp.float32),
            pltpu.VMEM((1, _D), jnp.float32),
            pltpu.SemaphoreType.DMA,
            pltpu.SemaphoreType.DMA,
        ],
    )
    return f(idx2d, tbl)


def kernel(Z, table):
    n, m = Z.shape
    B = n * m
    idx2d = Z.reshape(B // _CH, _CH).astype(jnp.int32)
    out = _gather(B, idx2d, table)
    return out.reshape(n, m, _D)
